# baseline (speedup 1.0000x reference)
import functools

import jax
import jax.numpy as jnp
from jax import lax
from jax.experimental import pallas as pl
from jax.experimental.pallas import tpu as pltpu

N = 4096
NFEAT = 512
NHID = 256
NCLASS = 40
NCPAD = 128
BM = 256
NBLK = N // BM


def _gcn_kernel(x_ref, w1_ref, w2_ref, adj_ref, logp_ref, h_ref,
                adjb_s, xw1_s, hw2_s):
    i = pl.program_id(0)

    @pl.when(i == 0)
    def _compute_xw1():
        xw1_s[...] = jnp.dot(
            x_ref[...], w1_ref[...],
            preferred_element_type=jnp.float32).astype(jnp.bfloat16)

    @pl.when(i < NBLK)
    def _layer1():
        ab = adj_ref[...].astype(jnp.bfloat16)
        adjb_s[pl.ds(i * BM, BM), :] = ab
        hb = jnp.maximum(
            jnp.dot(ab, xw1_s[...], preferred_element_type=jnp.float32), 0.0)
        h_ref[...] = hb
        hw2_s[pl.ds(i * BM, BM), :] = jnp.dot(
            hb.astype(jnp.bfloat16), w2_ref[...],
            preferred_element_type=jnp.float32).astype(jnp.bfloat16)

    @pl.when(i >= NBLK)
    def _layer2():
        j = i - NBLK
        z = jnp.dot(adjb_s[pl.ds(j * BM, BM), :], hw2_s[...],
                    preferred_element_type=jnp.float32)
        zr = jnp.maximum(z, 0.0)
        col = lax.broadcasted_iota(jnp.int32, (BM, NCPAD), 1)
        valid = col < NCLASS
        zm = jnp.where(valid, zr, -jnp.inf)
        m = jnp.max(zm, axis=1, keepdims=True)
        s = jnp.sum(jnp.where(valid, jnp.exp(zm - m), 0.0),
                    axis=1, keepdims=True)
        logp_ref[...] = (zr - m - jnp.log(s))[:, :NCLASS]


@functools.partial(jax.jit, static_argnames=())
def kernel(x, adj, W1, W2):
    w2p = jnp.pad(W2, ((0, 0), (0, NCPAD - NCLASS))).astype(jnp.bfloat16)
    grid = (2 * NBLK,)
    logp, h = pl.pallas_call(
        _gcn_kernel,
        grid=grid,
        in_specs=[
            pl.BlockSpec((N, NFEAT), lambda i: (0, 0)),
            pl.BlockSpec((NFEAT, NHID), lambda i: (0, 0)),
            pl.BlockSpec((NHID, NCPAD), lambda i: (0, 0)),
            pl.BlockSpec((BM, N), lambda i: (jnp.minimum(i, NBLK - 1), 0)),
        ],
        out_specs=[
            pl.BlockSpec((BM, NCLASS),
                         lambda i: (jnp.maximum(i - NBLK, 0), 0)),
            pl.BlockSpec((BM, NHID), lambda i: (jnp.minimum(i, NBLK - 1), 0)),
        ],
        out_shape=[
            jax.ShapeDtypeStruct((N, NCLASS), jnp.float32),
            jax.ShapeDtypeStruct((N, NHID), jnp.float32),
        ],
        scratch_shapes=[
            pltpu.VMEM((N, N), jnp.bfloat16),
            pltpu.VMEM((N, NHID), jnp.bfloat16),
            pltpu.VMEM((N, NCPAD), jnp.bfloat16),
        ],
        compiler_params=pltpu.CompilerParams(
            dimension_semantics=("arbitrary",),
        ),
    )(x, W1, w2p, adj)
    return (logp, h)


# speedup vs baseline: 1.0999x; 1.0999x over previous
import functools

import jax
import jax.numpy as jnp
from jax import lax
from jax.experimental import pallas as pl
from jax.experimental.pallas import tpu as pltpu

N = 4096
NFEAT = 512
NHID = 256
NCLASS = 40
NCPAD = 128
BM = 512
NBLK = N // BM


def _xw1_kernel(x_ref, w1_ref, xw1_ref):
    xw1_ref[...] = jnp.dot(
        x_ref[...], w1_ref[...],
        preferred_element_type=jnp.float32).astype(jnp.bfloat16)


def _gcn_kernel(xw1_ref, w2_ref, adj_ref, logp_ref, h_ref, adjb_s, hw2_s):
    i = pl.program_id(0)

    @pl.when(i < NBLK)
    def _layer1():
        ab = adj_ref[...].astype(jnp.bfloat16)
        adjb_s[pl.ds(i * BM, BM), :] = ab
        hb = jnp.maximum(
            jnp.dot(ab, xw1_ref[...], preferred_element_type=jnp.float32),
            0.0)
        h_ref[...] = hb
        hw2_s[pl.ds(i * BM, BM), :] = jnp.dot(
            hb.astype(jnp.bfloat16), w2_ref[...],
            preferred_element_type=jnp.float32).astype(jnp.bfloat16)

    @pl.when(i >= NBLK)
    def _layer2():
        j = i - NBLK
        z = jnp.dot(adjb_s[pl.ds(j * BM, BM), :], hw2_s[...],
                    preferred_element_type=jnp.float32)
        zr = jnp.maximum(z, 0.0)
        col = lax.broadcasted_iota(jnp.int32, (BM, NCPAD), 1)
        valid = col < NCLASS
        zm = jnp.where(valid, zr, -jnp.inf)
        m = jnp.max(zm, axis=1, keepdims=True)
        s = jnp.sum(jnp.where(valid, jnp.exp(zm - m), 0.0),
                    axis=1, keepdims=True)
        logp_ref[...] = (zr - m - jnp.log(s))[:, :NCLASS]


@functools.partial(jax.jit, static_argnames=())
def kernel(x, adj, W1, W2):
    w2p = jnp.pad(W2, ((0, 0), (0, NCPAD - NCLASS))).astype(jnp.bfloat16)
    xw1 = pl.pallas_call(
        _xw1_kernel,
        out_shape=jax.ShapeDtypeStruct((N, NHID), jnp.bfloat16),
    )(x, W1)
    grid = (2 * NBLK,)
    logp, h = pl.pallas_call(
        _gcn_kernel,
        grid=grid,
        in_specs=[
            pl.BlockSpec((N, NHID), lambda i: (0, 0)),
            pl.BlockSpec((NHID, NCPAD), lambda i: (0, 0)),
            pl.BlockSpec((BM, N), lambda i: (jnp.minimum(i, NBLK - 1), 0)),
        ],
        out_specs=[
            pl.BlockSpec((BM, NCLASS),
                         lambda i: (jnp.maximum(i - NBLK, 0), 0)),
            pl.BlockSpec((BM, NHID), lambda i: (jnp.minimum(i, NBLK - 1), 0)),
        ],
        out_shape=[
            jax.ShapeDtypeStruct((N, NCLASS), jnp.float32),
            jax.ShapeDtypeStruct((N, NHID), jnp.float32),
        ],
        scratch_shapes=[
            pltpu.VMEM((N, N), jnp.bfloat16),
            pltpu.VMEM((N, NCPAD), jnp.bfloat16),
        ],
        compiler_params=pltpu.CompilerParams(
            dimension_semantics=("arbitrary",),
        ),
    )(xw1, w2p, adj)
    return (logp, h)
